# Initial kernel scaffold; baseline (speedup 1.0000x reference)
import jax
import jax.numpy as jnp
from jax.experimental import pallas as pl


def kernel(x, edge_index, edge_weight, W1, b1, W2, b2):
    raise NotImplementedError("write your pallas kernel here")


# speedup vs baseline: 9.5269x; 9.5269x over previous
import functools

import jax
import jax.numpy as jnp
from jax import lax
from jax.experimental import pallas as pl
from jax.experimental.pallas import tpu as pltpu, tpu_sc as plsc

N_NODES = 10000
N_EDGES = 320000
D = 128

NC = 2
NS = 16
NW = NC * NS
CHUNK = 128
NCHUNK = -(-N_EDGES // (NW * CHUNK))
EPW = NCHUNK * CHUNK
E_PAD = NW * EPW
N_PAD = 10240
ROWS_PER_TILE = N_PAD // NS

_mesh = plsc.VectorSubcoreMesh(core_axis_name="c", subcore_axis_name="s")


@functools.partial(
    pl.kernel,
    out_type=jax.ShapeDtypeStruct((NC, N_PAD), jnp.float32),
    mesh=_mesh,
    scratch_types=[
        pltpu.VMEM((EPW,), jnp.int32),
        pltpu.VMEM((EPW,), jnp.float32),
        pltpu.VMEM((N_PAD,), jnp.float32),
        pltpu.VMEM((NS, ROWS_PER_TILE), jnp.float32),
        pltpu.VMEM((ROWS_PER_TILE,), jnp.float32),
        pltpu.VMEM_SHARED((NS, N_PAD), jnp.float32),
    ],
    compiler_params=pltpu.CompilerParams(needs_layout_passes=False),
)
def _deg_kernel(dst_hbm, ew_hbm, out_hbm, dst_v, ew_v, hist, red, res, stage):
    c = lax.axis_index("c")
    s = lax.axis_index("s")
    w = s * NC + c

    zeros16 = jnp.zeros((16,), jnp.float32)

    def zero_body(t, carry):
        hist[pl.ds(t * 16, 16)] = zeros16
        return carry

    lax.fori_loop(0, N_PAD // 16, zero_body, 0)

    pltpu.sync_copy(dst_hbm.at[pl.ds(w * EPW, EPW)], dst_v)
    pltpu.sync_copy(ew_hbm.at[pl.ds(w * EPW, EPW)], ew_v)

    def scat_body(t, carry):
        idx = dst_v[pl.ds(t * 16, 16)]
        vals = ew_v[pl.ds(t * 16, 16)]
        plsc.addupdate_scatter(hist, [idx], vals)
        return carry

    lax.fori_loop(0, EPW // 16, scat_body, 0)

    pltpu.sync_copy(hist, stage.at[s])
    plsc.subcore_barrier()
    for r in range(NS):
        pltpu.sync_copy(stage.at[r, pl.ds(s * ROWS_PER_TILE, ROWS_PER_TILE)],
                        red.at[r])

    def red_body(t, carry):
        acc = red[0, pl.ds(t * 16, 16)]
        for r in range(1, NS):
            acc = acc + red[r, pl.ds(t * 16, 16)]
        res[pl.ds(t * 16, 16)] = acc
        return carry

    lax.fori_loop(0, ROWS_PER_TILE // 16, red_body, 0)
    pltpu.sync_copy(res, out_hbm.at[c, pl.ds(s * ROWS_PER_TILE, ROWS_PER_TILE)])


@functools.partial(
    pl.kernel,
    out_type=jax.ShapeDtypeStruct((NC, N_PAD, D), jnp.float32),
    mesh=_mesh,
    scratch_types=[
        pltpu.VMEM((NCHUNK, CHUNK), jnp.int32),
        pltpu.VMEM((NCHUNK, CHUNK), jnp.int32),
        pltpu.VMEM((EPW,), jnp.float32),
        pltpu.VMEM((CHUNK, D), jnp.float32),
        pltpu.VMEM_SHARED((N_PAD, D), jnp.float32),
    ],
    compiler_params=pltpu.CompilerParams(needs_layout_passes=False),
)
def _agg_kernel(g_hbm, src_hbm, dst_hbm, ew_hbm, zeros_hbm, out_hbm,
                src_c, dst_c, ew_v, rows, acc):
    c = lax.axis_index("c")
    s = lax.axis_index("s")
    w = s * NC + c

    pltpu.sync_copy(zeros_hbm, acc.at[pl.ds(s * ROWS_PER_TILE, ROWS_PER_TILE)])

    pltpu.sync_copy(src_hbm.at[w], src_c)
    pltpu.sync_copy(dst_hbm.at[w], dst_c)
    pltpu.sync_copy(ew_hbm.at[pl.ds(w * EPW, EPW)], ew_v)
    plsc.subcore_barrier()

    def chunk_body(j, carry):
        pltpu.sync_copy(g_hbm.at[src_c.at[j]], rows)

        def row_body(r, carry2):
            ewb = plsc.load_gather(ew_v, [jnp.full((16,), j * CHUNK + r,
                                                   jnp.int32)])
            for k in range(D // 16):
                sl = pl.ds(k * 16, 16)
                rows[r, sl] = rows[r, sl] * ewb
            return carry2

        lax.fori_loop(0, CHUNK, row_body, 0)
        pltpu.sync_copy(rows, acc.at[dst_c.at[j]], add=True)
        return carry

    lax.fori_loop(0, NCHUNK, chunk_body, 0)

    plsc.subcore_barrier()
    pltpu.sync_copy(acc.at[pl.ds(s * ROWS_PER_TILE, ROWS_PER_TILE)],
                    out_hbm.at[c, pl.ds(s * ROWS_PER_TILE, ROWS_PER_TILE)])


_BLK = 1024


def _tc1_body(x_ref, w_ref, degp_ref, g_ref, dis_ref):
    h = jnp.dot(x_ref[...], w_ref[...], preferred_element_type=jnp.float32)
    deg = degp_ref[:, 0:1] + degp_ref[:, 1:2] + 1.0
    dis = lax.rsqrt(deg)
    g_ref[...] = h * dis
    dis_ref[...] = dis


def _tc1(x_pad, W1, degp_t):
    grid = (N_PAD // _BLK,)
    return pl.pallas_call(
        _tc1_body,
        grid=grid,
        in_specs=[
            pl.BlockSpec((_BLK, D), lambda i: (i, 0)),
            pl.BlockSpec((D, D), lambda i: (0, 0)),
            pl.BlockSpec((_BLK, NC), lambda i: (i, 0)),
        ],
        out_specs=[
            pl.BlockSpec((_BLK, D), lambda i: (i, 0)),
            pl.BlockSpec((_BLK, 1), lambda i: (i, 0)),
        ],
        out_shape=[
            jax.ShapeDtypeStruct((N_PAD, D), jnp.float32),
            jax.ShapeDtypeStruct((N_PAD, 1), jnp.float32),
        ],
    )(x_pad, W1, degp_t)


def _tc2_body(a0_ref, a1_ref, g1_ref, dis_ref, b1_ref, w2_ref, g2_ref):
    tot = a0_ref[...] + a1_ref[...] + g1_ref[...]
    z = jnp.maximum(tot * dis_ref[...] + b1_ref[...], 0.0)
    h2 = jnp.dot(z, w2_ref[...], preferred_element_type=jnp.float32)
    g2_ref[...] = h2 * dis_ref[...]


def _tc2(a0, a1, g1, dis, b1, W2):
    grid = (N_PAD // _BLK,)
    return pl.pallas_call(
        _tc2_body,
        grid=grid,
        in_specs=[
            pl.BlockSpec((_BLK, D), lambda i: (i, 0)),
            pl.BlockSpec((_BLK, D), lambda i: (i, 0)),
            pl.BlockSpec((_BLK, D), lambda i: (i, 0)),
            pl.BlockSpec((_BLK, 1), lambda i: (i, 0)),
            pl.BlockSpec((1, D), lambda i: (0, 0)),
            pl.BlockSpec((D, D), lambda i: (0, 0)),
        ],
        out_specs=pl.BlockSpec((_BLK, D), lambda i: (i, 0)),
        out_shape=jax.ShapeDtypeStruct((N_PAD, D), jnp.float32),
    )(a0, a1, g1, dis, b1, W2)


def _tc3_body(a0_ref, a1_ref, g2_ref, dis_ref, b2_ref, out_ref):
    tot = a0_ref[...] + a1_ref[...] + g2_ref[...]
    out_ref[...] = tot * dis_ref[...] + b2_ref[...]


def _tc3(a0, a1, g2, dis, b2):
    grid = (N_PAD // _BLK,)
    return pl.pallas_call(
        _tc3_body,
        grid=grid,
        in_specs=[
            pl.BlockSpec((_BLK, D), lambda i: (i, 0)),
            pl.BlockSpec((_BLK, D), lambda i: (i, 0)),
            pl.BlockSpec((_BLK, D), lambda i: (i, 0)),
            pl.BlockSpec((_BLK, 1), lambda i: (i, 0)),
            pl.BlockSpec((1, D), lambda i: (0, 0)),
        ],
        out_specs=pl.BlockSpec((_BLK, D), lambda i: (i, 0)),
        out_shape=jax.ShapeDtypeStruct((N_PAD, D), jnp.float32),
    )(a0, a1, g2, dis, b2)


def kernel(x, edge_index, edge_weight, W1, b1, W2, b2):
    src = edge_index[0].astype(jnp.int32)
    dst = edge_index[1].astype(jnp.int32)
    ew = edge_weight.astype(jnp.float32)

    pad_e = E_PAD - N_EDGES
    src_p = jnp.pad(src, (0, pad_e))
    dst_p = jnp.pad(dst, (0, pad_e))
    ew_p = jnp.pad(ew, (0, pad_e))
    src3 = src_p.reshape(NW, NCHUNK, CHUNK)
    dst3 = dst_p.reshape(NW, NCHUNK, CHUNK)

    x_pad = jnp.pad(x, ((0, N_PAD - N_NODES), (0, 0)))
    zeros_blk = jnp.zeros((ROWS_PER_TILE, D), jnp.float32)

    degp = _deg_kernel(dst_p, ew_p)
    g1, dis = _tc1(x_pad, W1, degp.T)
    acc1 = _agg_kernel(g1, src3, dst3, ew_p, zeros_blk)
    g2 = _tc2(acc1[0], acc1[1], g1, dis, b1.reshape(1, D), W2)
    acc2 = _agg_kernel(g2, src3, dst3, ew_p, zeros_blk)
    out = _tc3(acc2[0], acc2[1], g2, dis, b2.reshape(1, D))
    return out[:N_NODES]
